# baseline (speedup 1.0000x reference)
import jax
import jax.numpy as jnp
from jax import lax
from jax.experimental import pallas as pl
from jax.experimental.pallas import tpu as pltpu
from jax.experimental.pallas import tpu_sc as plsc

GRAPH_NUM = 3
D = 128
NSAMP = 8192
HALF = NSAMP // 2
LEAKY = 0.2

_NC = 2
_NS = 16
_L = 16
_NW = _NC * _NS
_BPW = NSAMP // _NW
_NBUF = 3


def _leaky(x):
    return jnp.where(x > 0, x, LEAKY * x)


def _sc_gather_all(fu, uvf, fi, ivf, su, si, n_users, n_items):
    def body(fu_hbm, uvf_hbm, fi_hbm, ivf_hbm, su_hbm, si_hbm,
             fug, uvg, fig, ivg,
             isu0, isu1, isu2, isi0, isi1, isi2, iuv1, iuv2, iiv1, iiv2,
             rows0, rows1, rows2,
             isem, gsem0, gsem1, gsem2, ssem0, ssem1, ssem2):
        wid = lax.axis_index("s") * _NC + lax.axis_index("c")
        base = wid * _BPW
        rows = (rows0, rows1, rows2)
        gsems = (gsem0, gsem1, gsem2)
        ssems = (ssem0, ssem1, ssem2)
        isu = (isu0, isu1, isu2)
        isi = (isi0, isi1, isi2)

        loads = []
        for g in range(GRAPH_NUM):
            loads.append(pltpu.async_copy(
                su_hbm.at[pl.ds(g * NSAMP + base, _BPW)], isu[g], isem))
            loads.append(pltpu.async_copy(
                si_hbm.at[pl.ds(g * NSAMP + base, _BPW)], isi[g], isem))
        for c in loads:
            c.wait()

        for dst, srcv, off in ((iuv1, isu1, n_users), (iuv2, isu2, 2 * n_users),
                               (iiv1, isi1, n_items), (iiv2, isi2, 2 * n_items)):
            for k in range(_BPW // _L):
                sl = pl.ds(k * _L, _L)
                dst[sl] = srcv[sl] + off

        uv_idx = (isu0, iuv1, iuv2)
        iv_idx = (isi0, iiv1, iiv2)
        rounds = []
        for g in range(GRAPH_NUM):
            ob = g * NSAMP + base
            rounds.append((fu_hbm, isu[g], fug, ob))
            rounds.append((uvf_hbm, uv_idx[g], uvg, ob))
            rounds.append((fi_hbm, isi[g], fig, ob))
            rounds.append((ivf_hbm, iv_idx[g], ivg, ob))

        nr = len(rounds)
        gathers = [None] * nr
        scatters = [None] * nr

        def start_gather(r):
            tab, iref, _, _ = rounds[r]
            b = r % _NBUF
            gathers[r] = pltpu.async_copy(tab.at[iref], rows[b], gsems[b])

        def retire(r):
            _, _, out_ref, ob = rounds[r]
            b = r % _NBUF
            gathers[r].wait()
            scatters[r] = pltpu.async_copy(rows[b],
                                           out_ref.at[pl.ds(ob, _BPW)],
                                           ssems[b])

        for r in range(_NBUF):
            start_gather(r)
        for r in range(_NBUF, nr):
            retire(r - _NBUF)
            scatters[r - _NBUF].wait()
            start_gather(r)
        for r in range(nr - _NBUF, nr):
            retire(r)
        for r in range(nr - _NBUF, nr):
            scatters[r].wait()

    out = jax.ShapeDtypeStruct((GRAPH_NUM * NSAMP, D), jnp.float32)
    kern = pl.kernel(
        body,
        out_type=[out, out, out, out],
        mesh=plsc.VectorSubcoreMesh(core_axis_name="c", subcore_axis_name="s"),
        scratch_types=[
            pltpu.VMEM((_BPW,), jnp.int32),
            pltpu.VMEM((_BPW,), jnp.int32),
            pltpu.VMEM((_BPW,), jnp.int32),
            pltpu.VMEM((_BPW,), jnp.int32),
            pltpu.VMEM((_BPW,), jnp.int32),
            pltpu.VMEM((_BPW,), jnp.int32),
            pltpu.VMEM((_BPW,), jnp.int32),
            pltpu.VMEM((_BPW,), jnp.int32),
            pltpu.VMEM((_BPW,), jnp.int32),
            pltpu.VMEM((_BPW,), jnp.int32),
            pltpu.VMEM((_BPW, D), jnp.float32),
            pltpu.VMEM((_BPW, D), jnp.float32),
            pltpu.VMEM((_BPW, D), jnp.float32),
            pltpu.SemaphoreType.DMA,
            pltpu.SemaphoreType.DMA,
            pltpu.SemaphoreType.DMA,
            pltpu.SemaphoreType.DMA,
            pltpu.SemaphoreType.DMA,
            pltpu.SemaphoreType.DMA,
            pltpu.SemaphoreType.DMA,
        ],
    )
    return kern(fu, uvf, fi, ivf, su, si)


def _tc_body(fu_p, fu_n, uv_p, uv_n, fi_p, fi_n, iv_p, iv_n,
             w1, b1, w2, b2, out):
    @pl.when((pl.program_id(0) == 0) & (pl.program_id(1) == 0))
    def _():
        out[...] = jnp.zeros_like(out)

    W1 = w1[...]
    b1v = b1[...]
    w2v = w2[...]
    b2s = b2[0, 0]

    def weight(fu, uv):
        h = (jnp.dot(fu * uv, W1[:D], preferred_element_type=jnp.float32)
             + jnp.dot(fu, W1[D:2 * D], preferred_element_type=jnp.float32)
             + jnp.dot(uv, W1[2 * D:], preferred_element_type=jnp.float32)
             + b1v)
        h = _leaky(h)
        z = jnp.sum(h * w2v, axis=-1) + b2s
        return 1.0 / (1.0 + jnp.exp(-z))

    fu_pv, uv_pv = fu_p[...], uv_p[...]
    fu_nv, uv_nv = fu_n[...], uv_n[...]
    wpos = weight(fu_pv, uv_pv)
    wneg = weight(fu_nv, uv_nv)
    spos = jnp.sum(_leaky(fu_pv * fi_p[...]), axis=-1)
    sneg = jnp.sum(_leaky(fu_nv * fi_n[...]), axis=-1)
    ppos = jnp.sum(_leaky(uv_pv * iv_p[...]), axis=-1)
    pneg = jnp.sum(_leaky(uv_nv * iv_n[...]), axis=-1)
    s = wpos * spos - wneg * sneg
    l = jnp.sum(jnp.maximum(0.0, 1.0 - s * (ppos - pneg)))
    out[...] = out[...] + l


def _tc_loss(fug, uvg, fig, ivg, w1, b1r, w2r, b2r):
    BP = 4096
    nbj = HALF // BP
    nbg = NSAMP // BP

    rs_p = pl.BlockSpec((BP, D), lambda i, j: (i * nbg + j, 0))
    rs_n = pl.BlockSpec((BP, D), lambda i, j: (i * nbg + nbj + j, 0))

    def full(shape):
        return pl.BlockSpec(shape, lambda i, j: (0, 0))

    out = pl.pallas_call(
        _tc_body,
        grid=(GRAPH_NUM, nbj),
        in_specs=[rs_p, rs_n, rs_p, rs_n, rs_p, rs_n, rs_p, rs_n,
                  full((3 * D, D)), full((1, D)), full((1, D)), full((1, 1))],
        out_specs=pl.BlockSpec((1, 1), lambda i, j: (0, 0)),
        out_shape=jax.ShapeDtypeStruct((1, 1), jnp.float32),
    )(fug, fug, uvg, uvg, fig, fig, ivg, ivg, w1, b1r, w2r, b2r)
    return out[0, 0]


def kernel(final_user_vector, user_vector, final_item_vector, item_vector,
           suids0, suids1, suids2, siids0, siids1, siids2, W1, b1, W2, b2):
    n_users = final_user_vector.shape[0]
    n_items = final_item_vector.shape[0]
    su = jnp.concatenate([suids0, suids1, suids2]).astype(jnp.int32)
    si = jnp.concatenate([siids0, siids1, siids2]).astype(jnp.int32)
    uvf = user_vector.reshape(GRAPH_NUM * n_users, D)
    ivf = item_vector.reshape(GRAPH_NUM * n_items, D)
    fug, uvg, fig, ivg = _sc_gather_all(
        final_user_vector, uvf, final_item_vector, ivf, su, si,
        n_users, n_items)
    return _tc_loss(fug, uvg, fig, ivg, W1,
                    b1.reshape(1, D), W2.reshape(1, D), b2.reshape(1, 1))
